# baseline (speedup 1.0000x reference)
import functools

import jax
import jax.numpy as jnp
from jax import lax
from jax.experimental import pallas as pl
from jax.experimental.pallas import tpu as pltpu
from jax.experimental.pallas import tpu_sc as plsc

N = 10000
E = 320000
IN_DIM = 16
HID = 128
HH = HID // 2

NC = 2
NS = 16

CHUNK = 128
EPT = E // NS
NCH = EPT // CHUNK
TAIL = EPT - NCH * CHUNK
ACC_R = 10112
RPT = ACC_R // NS
NBUF = 4
G = 2


RC = 2000


def _sc_aggregate(h_split, edges):
    mesh = plsc.VectorSubcoreMesh(core_axis_name="c", subcore_axis_name="s")

    @functools.partial(
        pl.kernel,
        out_type=(
            jax.ShapeDtypeStruct((NC, N, HH), jnp.float32),
            jax.ShapeDtypeStruct((N // RC, NS, RC), jnp.float32),
        ),
        mesh=mesh,
        scratch_types=(
            pltpu.VMEM((EPT,), jnp.int32),
            pltpu.VMEM((EPT,), jnp.int32),
            pltpu.VMEM((NBUF, CHUNK, HH), jnp.float32),
            pltpu.VMEM((ACC_R,), jnp.float32),
            pltpu.VMEM_SHARED((ACC_R, HH), jnp.float32),
            pltpu.SemaphoreType.DMA,
            pltpu.SemaphoreType.DMA,
        ),
        compiler_params=pltpu.CompilerParams(needs_layout_passes=False,
                                             use_tc_tiling_on_sc=False),
    )
    def agg(h_hbm, edges_hbm, msum_out, deg_out,
            srcv, dstv, rows, degl, acc, gsem, ssem):
        cid = lax.axis_index("c")
        sid = lax.axis_index("s")

        eb = sid * EPT
        pltpu.sync_copy(edges_hbm.at[0, pl.ds(eb, EPT)], srcv)
        pltpu.sync_copy(edges_hbm.at[1, pl.ds(eb, EPT)], dstv)

        def zero_rows(i, _):
            r = i // (HH // 16)
            c = i % (HH // 16)
            rows[0, r, pl.ds(c * 16, 16)] = jnp.zeros((16,), jnp.float32)
            return 0
        lax.fori_loop(0, CHUNK * (HH // 16), zero_rows, 0)

        zb = sid * RPT
        off = 0
        while off < RPT:
            sz = min(CHUNK, RPT - off)
            pltpu.sync_copy(rows.at[0, pl.ds(0, sz)],
                            acc.at[pl.ds(zb + off, sz)])
            off += sz
        plsc.subcore_barrier()

        ones16 = jnp.ones((16,), jnp.float32)
        h_c = h_hbm.at[cid]

        def g_fire(j):
            jm = lax.rem(j, NCH)
            pltpu.async_copy(h_c.at[srcv.at[pl.ds(jm * CHUNK, CHUNK)]],
                             rows.at[lax.rem(j, NBUF)], gsem)

        def g_wait(j):
            jm = lax.rem(j, NCH)
            pltpu.make_async_copy(h_c.at[srcv.at[pl.ds(jm * CHUNK, CHUNK)]],
                                  rows.at[lax.rem(j, NBUF)], gsem).wait()

        def s_fire(j):
            jm = lax.rem(j, NCH)
            pltpu.async_copy(rows.at[lax.rem(j, NBUF)],
                             acc.at[dstv.at[pl.ds(jm * CHUNK, CHUNK)]],
                             ssem, add=True)

        def s_wait(j):
            jm = lax.rem(j, NCH)
            pltpu.make_async_copy(rows.at[lax.rem(j, NBUF)],
                                  acc.at[dstv.at[pl.ds(jm * CHUNK, CHUNK)]],
                                  ssem).wait()

        for j in range(G):
            g_fire(j)

        def zero_deg(i, _):
            degl[pl.ds(i * 16, 16)] = jnp.zeros((16,), jnp.float32)
            return 0
        lax.fori_loop(0, ACC_R // 16, zero_deg, 0)

        def chunk_body(j, _):
            g_wait(j)

            @pl.when(j >= NBUF - G)
            def _():
                s_wait(j - (NBUF - G))

            @pl.when(j + G < NCH)
            def _():
                g_fire(j + G)

            s_fire(j)

            def deg_body(i, _):
                d16 = dstv[pl.ds(j * CHUNK + i * 16, 16)]
                plsc.addupdate_scatter(degl, [d16], ones16)
                return 0
            lax.fori_loop(0, CHUNK // 16, deg_body, 0)
            return 0

        lax.fori_loop(0, NCH, chunk_body, 0)
        for j in range(NCH - (NBUF - G), NCH):
            s_wait(j)

        tb = NCH * CHUNK
        pltpu.async_copy(h_c.at[srcv.at[pl.ds(tb, TAIL)]],
                         rows.at[0, pl.ds(0, TAIL)], gsem)
        pltpu.make_async_copy(h_c.at[srcv.at[pl.ds(tb, TAIL)]],
                              rows.at[0, pl.ds(0, TAIL)], gsem).wait()
        pltpu.async_copy(rows.at[0, pl.ds(0, TAIL)],
                         acc.at[dstv.at[pl.ds(tb, TAIL)]], ssem, add=True)
        pltpu.make_async_copy(rows.at[0, pl.ds(0, TAIL)],
                              acc.at[dstv.at[pl.ds(tb, TAIL)]], ssem).wait()

        def tail_deg(i, _):
            d16 = dstv[pl.ds(tb + i * 16, 16)]
            plsc.addupdate_scatter(degl, [d16], ones16)
            return 0
        lax.fori_loop(0, TAIL // 16, tail_deg, 0)
        plsc.subcore_barrier()

        ob = sid * 624
        pltpu.sync_copy(acc.at[pl.ds(ob, 624)],
                        msum_out.at[cid, pl.ds(ob, 624)])

        @pl.when(sid == NS - 1)
        def _():
            pltpu.sync_copy(acc.at[pl.ds(16 * 624, N - 16 * 624)],
                            msum_out.at[cid, pl.ds(16 * 624, N - 16 * 624)])

        @pl.when(cid == 0)
        def _():
            for k in range(N // RC):
                pltpu.sync_copy(degl.at[pl.ds(k * RC, RC)],
                                deg_out.at[k, sid])

    return agg(h_split, edges)


def _tc_init(features, W_init, b_init):
    R = 2000

    def body(x_ref, w_ref, b_ref, o_ref, oc_ref):
        y = jnp.dot(x_ref[...], w_ref[...], preferred_element_type=jnp.float32)
        y = jnp.maximum(y + b_ref[...], 0.0)
        o_ref[...] = y
        oc_ref[0] = y[:, :HH]
        oc_ref[1] = y[:, HH:]

    return pl.pallas_call(
        body,
        grid=(N // R,),
        in_specs=[
            pl.BlockSpec((R, IN_DIM), lambda i: (i, 0)),
            pl.BlockSpec((IN_DIM, HID), lambda i: (0, 0)),
            pl.BlockSpec((1, HID), lambda i: (0, 0)),
        ],
        out_specs=[
            pl.BlockSpec((R, HID), lambda i: (i, 0)),
            pl.BlockSpec((NC, R, HH), lambda i: (0, i, 0)),
        ],
        out_shape=[
            jax.ShapeDtypeStruct((N, HID), jnp.float32),
            jax.ShapeDtypeStruct((NC, N, HH), jnp.float32),
        ],
    )(features, W_init, b_init.reshape(1, HID))


def _tc_combine(h, msum, deg_t, W_self, b_self, W_neigh, b_neigh, act,
                split_out):
    R = RC

    def body(h_ref, m_ref, d_ref, ws_ref, wn_ref, bs_ref, bn_ref, *outs):
        h_blk = h_ref[...]
        msum_blk = jnp.concatenate([m_ref[0], m_ref[1]], axis=1)
        deg = jnp.sum(d_ref[0], axis=0)
        h_neigh = msum_blk / jnp.clip(deg, 1.0)[:, None]
        out = (jnp.dot(h_blk, ws_ref[...], preferred_element_type=jnp.float32)
               + jnp.dot(h_neigh, wn_ref[...],
                         preferred_element_type=jnp.float32)
               + bs_ref[...] + bn_ref[...])
        if act:
            out = jnp.maximum(out, 0.0)
        outs[0][...] = out
        if split_out:
            outs[1][0] = out[:, :HH]
            outs[1][1] = out[:, HH:]

    out_specs = [pl.BlockSpec((R, HID), lambda i: (i, 0))]
    out_shape = [jax.ShapeDtypeStruct((N, HID), jnp.float32)]
    if split_out:
        out_specs.append(pl.BlockSpec((NC, R, HH), lambda i: (0, i, 0)))
        out_shape.append(jax.ShapeDtypeStruct((NC, N, HH), jnp.float32))

    return pl.pallas_call(
        body,
        grid=(N // R,),
        in_specs=[
            pl.BlockSpec((R, HID), lambda i: (i, 0)),
            pl.BlockSpec((NC, R, HH), lambda i: (0, i, 0)),
            pl.BlockSpec((1, NS, R), lambda i: (i, 0, 0)),
            pl.BlockSpec((HID, HID), lambda i: (0, 0)),
            pl.BlockSpec((HID, HID), lambda i: (0, 0)),
            pl.BlockSpec((1, HID), lambda i: (0, 0)),
            pl.BlockSpec((1, HID), lambda i: (0, 0)),
        ],
        out_specs=out_specs,
        out_shape=out_shape,
    )(h, msum, deg_t, W_self, W_neigh,
      b_self.reshape(1, HID), b_neigh.reshape(1, HID))


def kernel(features, edge_index0, edge_index1, W_init, b_init,
           W_self, b_self, W_neigh, b_neigh):
    e0 = edge_index0.astype(jnp.int32)
    e1 = edge_index1.astype(jnp.int32)

    h, hc = _tc_init(features, W_init, b_init)
    msum, deg_p = _sc_aggregate(hc, e0)
    h, hc = _tc_combine(h, msum, deg_p, W_self, b_self,
                        W_neigh, b_neigh, act=True, split_out=True)
    msum, deg_p = _sc_aggregate(hc, e1)
    (h,) = _tc_combine(h, msum, deg_p, W_self, b_self,
                       W_neigh, b_neigh, act=False, split_out=False)
    return h


# speedup vs baseline: 1.0814x; 1.0814x over previous
import functools

import jax
import jax.numpy as jnp
from jax import lax
from jax.experimental import pallas as pl
from jax.experimental.pallas import tpu as pltpu
from jax.experimental.pallas import tpu_sc as plsc

N = 10000
E = 320000
IN_DIM = 16
HID = 128
HH = HID // 2

NC = 2
NS = 16

CHUNK = 128
EPT = E // NS
NCH = EPT // CHUNK
TAIL = EPT - NCH * CHUNK
ACC_R = 10112
RPT = ACC_R // NS
NBUF = 4
G = 3


RC = 2000


def _sc_aggregate(h_split, edges):
    mesh = plsc.VectorSubcoreMesh(core_axis_name="c", subcore_axis_name="s")

    @functools.partial(
        pl.kernel,
        out_type=(
            jax.ShapeDtypeStruct((NC, N, HH), jnp.float32),
            jax.ShapeDtypeStruct((N // RC, NS, RC), jnp.float32),
        ),
        mesh=mesh,
        scratch_types=(
            pltpu.VMEM((EPT,), jnp.int32),
            pltpu.VMEM((EPT,), jnp.int32),
            pltpu.VMEM((NBUF, CHUNK, HH), jnp.float32),
            pltpu.VMEM((ACC_R,), jnp.float32),
            pltpu.VMEM_SHARED((ACC_R, HH), jnp.float32),
            pltpu.SemaphoreType.DMA,
            pltpu.SemaphoreType.DMA,
        ),
        compiler_params=pltpu.CompilerParams(needs_layout_passes=False,
                                             use_tc_tiling_on_sc=False),
    )
    def agg(h_hbm, edges_hbm, msum_out, deg_out,
            srcv, dstv, rows, degl, acc, gsem, ssem):
        cid = lax.axis_index("c")
        sid = lax.axis_index("s")

        eb = sid * EPT
        pltpu.sync_copy(edges_hbm.at[0, pl.ds(eb, EPT)], srcv)
        pltpu.sync_copy(edges_hbm.at[1, pl.ds(eb, EPT)], dstv)

        def zero_rows(i, _):
            r = i // (HH // 16)
            c = i % (HH // 16)
            rows[0, r, pl.ds(c * 16, 16)] = jnp.zeros((16,), jnp.float32)
            return 0
        lax.fori_loop(0, CHUNK * (HH // 16), zero_rows, 0)

        zb = sid * RPT
        off = 0
        while off < RPT:
            sz = min(CHUNK, RPT - off)
            pltpu.sync_copy(rows.at[0, pl.ds(0, sz)],
                            acc.at[pl.ds(zb + off, sz)])
            off += sz
        plsc.subcore_barrier()

        ones16 = jnp.ones((16,), jnp.float32)
        h_c = h_hbm.at[cid]

        def g_fire(j):
            jm = lax.rem(j, NCH)
            pltpu.async_copy(h_c.at[srcv.at[pl.ds(jm * CHUNK, CHUNK)]],
                             rows.at[lax.rem(j, NBUF)], gsem)

        def g_wait(j):
            jm = lax.rem(j, NCH)
            pltpu.make_async_copy(h_c.at[srcv.at[pl.ds(jm * CHUNK, CHUNK)]],
                                  rows.at[lax.rem(j, NBUF)], gsem).wait()

        def s_fire(j):
            jm = lax.rem(j, NCH)
            pltpu.async_copy(rows.at[lax.rem(j, NBUF)],
                             acc.at[dstv.at[pl.ds(jm * CHUNK, CHUNK)]],
                             ssem, add=True)

        def s_wait(j):
            jm = lax.rem(j, NCH)
            pltpu.make_async_copy(rows.at[lax.rem(j, NBUF)],
                                  acc.at[dstv.at[pl.ds(jm * CHUNK, CHUNK)]],
                                  ssem).wait()

        for j in range(G):
            g_fire(j)

        def zero_deg(i, _):
            degl[pl.ds(i * 16, 16)] = jnp.zeros((16,), jnp.float32)
            return 0
        lax.fori_loop(0, ACC_R // 16, zero_deg, 0)

        def chunk_body(j, _):
            g_wait(j)

            @pl.when(j >= NBUF - G)
            def _():
                s_wait(j - (NBUF - G))

            @pl.when(j + G < NCH)
            def _():
                g_fire(j + G)

            s_fire(j)

            def deg_body(i, _):
                d16 = dstv[pl.ds(j * CHUNK + i * 16, 16)]
                plsc.addupdate_scatter(degl, [d16], ones16)
                return 0
            lax.fori_loop(0, CHUNK // 16, deg_body, 0)
            return 0

        lax.fori_loop(0, NCH, chunk_body, 0)
        for j in range(NCH - (NBUF - G), NCH):
            s_wait(j)

        tb = NCH * CHUNK
        pltpu.async_copy(h_c.at[srcv.at[pl.ds(tb, TAIL)]],
                         rows.at[0, pl.ds(0, TAIL)], gsem)
        pltpu.make_async_copy(h_c.at[srcv.at[pl.ds(tb, TAIL)]],
                              rows.at[0, pl.ds(0, TAIL)], gsem).wait()
        pltpu.async_copy(rows.at[0, pl.ds(0, TAIL)],
                         acc.at[dstv.at[pl.ds(tb, TAIL)]], ssem, add=True)
        pltpu.make_async_copy(rows.at[0, pl.ds(0, TAIL)],
                              acc.at[dstv.at[pl.ds(tb, TAIL)]], ssem).wait()

        def tail_deg(i, _):
            d16 = dstv[pl.ds(tb + i * 16, 16)]
            plsc.addupdate_scatter(degl, [d16], ones16)
            return 0
        lax.fori_loop(0, TAIL // 16, tail_deg, 0)
        plsc.subcore_barrier()

        ob = sid * 624
        pltpu.sync_copy(acc.at[pl.ds(ob, 624)],
                        msum_out.at[cid, pl.ds(ob, 624)])

        @pl.when(sid == NS - 1)
        def _():
            pltpu.sync_copy(acc.at[pl.ds(16 * 624, N - 16 * 624)],
                            msum_out.at[cid, pl.ds(16 * 624, N - 16 * 624)])

        @pl.when(cid == 0)
        def _():
            for k in range(N // RC):
                pltpu.sync_copy(degl.at[pl.ds(k * RC, RC)],
                                deg_out.at[k, sid])

    return agg(h_split, edges)


def _tc_init(features, W_init, b_init):
    R = 2000

    def body(x_ref, w_ref, b_ref, o_ref, oc_ref):
        y = jnp.dot(x_ref[...], w_ref[...], preferred_element_type=jnp.float32)
        y = jnp.maximum(y + b_ref[...], 0.0)
        o_ref[...] = y
        oc_ref[0] = y[:, :HH]
        oc_ref[1] = y[:, HH:]

    return pl.pallas_call(
        body,
        grid=(N // R,),
        in_specs=[
            pl.BlockSpec((R, IN_DIM), lambda i: (i, 0)),
            pl.BlockSpec((IN_DIM, HID), lambda i: (0, 0)),
            pl.BlockSpec((1, HID), lambda i: (0, 0)),
        ],
        out_specs=[
            pl.BlockSpec((R, HID), lambda i: (i, 0)),
            pl.BlockSpec((NC, R, HH), lambda i: (0, i, 0)),
        ],
        out_shape=[
            jax.ShapeDtypeStruct((N, HID), jnp.float32),
            jax.ShapeDtypeStruct((NC, N, HH), jnp.float32),
        ],
    )(features, W_init, b_init.reshape(1, HID))


def _tc_combine(h, msum, deg_t, W_self, b_self, W_neigh, b_neigh, act,
                split_out):
    R = RC

    def body(h_ref, m_ref, d_ref, ws_ref, wn_ref, bs_ref, bn_ref, *outs):
        h_blk = h_ref[...]
        msum_blk = jnp.concatenate([m_ref[0], m_ref[1]], axis=1)
        deg = jnp.sum(d_ref[0], axis=0)
        h_neigh = msum_blk / jnp.clip(deg, 1.0)[:, None]
        out = (jnp.dot(h_blk, ws_ref[...], preferred_element_type=jnp.float32)
               + jnp.dot(h_neigh, wn_ref[...],
                         preferred_element_type=jnp.float32)
               + bs_ref[...] + bn_ref[...])
        if act:
            out = jnp.maximum(out, 0.0)
        outs[0][...] = out
        if split_out:
            outs[1][0] = out[:, :HH]
            outs[1][1] = out[:, HH:]

    out_specs = [pl.BlockSpec((R, HID), lambda i: (i, 0))]
    out_shape = [jax.ShapeDtypeStruct((N, HID), jnp.float32)]
    if split_out:
        out_specs.append(pl.BlockSpec((NC, R, HH), lambda i: (0, i, 0)))
        out_shape.append(jax.ShapeDtypeStruct((NC, N, HH), jnp.float32))

    return pl.pallas_call(
        body,
        grid=(N // R,),
        in_specs=[
            pl.BlockSpec((R, HID), lambda i: (i, 0)),
            pl.BlockSpec((NC, R, HH), lambda i: (0, i, 0)),
            pl.BlockSpec((1, NS, R), lambda i: (i, 0, 0)),
            pl.BlockSpec((HID, HID), lambda i: (0, 0)),
            pl.BlockSpec((HID, HID), lambda i: (0, 0)),
            pl.BlockSpec((1, HID), lambda i: (0, 0)),
            pl.BlockSpec((1, HID), lambda i: (0, 0)),
        ],
        out_specs=out_specs,
        out_shape=out_shape,
    )(h, msum, deg_t, W_self, W_neigh,
      b_self.reshape(1, HID), b_neigh.reshape(1, HID))


def kernel(features, edge_index0, edge_index1, W_init, b_init,
           W_self, b_self, W_neigh, b_neigh):
    e0 = edge_index0.astype(jnp.int32)
    e1 = edge_index1.astype(jnp.int32)

    h, hc = _tc_init(features, W_init, b_init)
    msum, deg_p = _sc_aggregate(hc, e0)
    h, hc = _tc_combine(h, msum, deg_p, W_self, b_self,
                        W_neigh, b_neigh, act=True, split_out=True)
    msum, deg_p = _sc_aggregate(hc, e1)
    (h,) = _tc_combine(h, msum, deg_p, W_self, b_self,
                       W_neigh, b_neigh, act=False, split_out=False)
    return h
